# baseline (speedup 1.0000x reference)
import jax
import jax.numpy as jnp
from jax.experimental import pallas as pl
from jax.experimental.pallas import tpu as pltpu

_B, _L, _D = 4, 128, 768
_N = _B * _L * _L
_CHUNK = 8192
_NCHUNK = _N // _CHUNK
_CPB = _N // _B // _CHUNK
_ONE_BITS_P1 = 0x3F800001


def _phase1_body(w8_ref, bias_ref, x_ref, ys_ref, ye_ref,
                 zs_ref, ze_ref, accs_ref, acce_ref):
    i = pl.program_id(0)
    x = x_ref[...]
    wcat = w8_ref[...]
    x_hi = x.astype(jnp.bfloat16)
    dn = (((1,), (0,)), ((), ()))
    res = jax.lax.dot_general(
        x_hi, wcat, dn,
        preferred_element_type=jnp.float32)
    rt = res.T
    zs = rt[0:1, :] + bias_ref[0]
    ze = rt[1:2, :] + bias_ref[1]
    ys_i = ys_ref[0]
    ye_i = ye_ref[0]
    ys = ys_i.astype(jnp.float32)
    ye = ye_i.astype(jnp.float32)
    w = (ys_i >= 0).astype(jnp.float32)
    bce_s = jnp.maximum(zs, 0.0) - zs * ys + jnp.log1p(jnp.exp(-jnp.abs(zs)))
    bce_e = jnp.maximum(ze, 0.0) - ze * ye + jnp.log1p(jnp.exp(-jnp.abs(ze)))

    @pl.when(i == 0)
    def _():
        accs_ref[...] = jnp.zeros_like(accs_ref)
        acce_ref[...] = jnp.zeros_like(acce_ref)

    accs_ref[...] += (w * bce_s)[None]
    acce_ref[...] += (w * bce_e)[None]
    neg = jnp.float32(-jnp.inf)
    zs_ref[...] = jnp.where(w > 0, zs, neg)[None]
    ze_ref[...] = jnp.where(w > 0, ze, neg)[None]


def _phase2_body(am_ref, zs_ref, ze_ref, accs_ref, acce_ref,
                 ms_ref, me_ref, ls_ref, le_ref):
    b = pl.program_id(0)
    mask_len = jnp.sum(am_ref[...]) - 2
    k = jnp.maximum((mask_len.astype(jnp.float32) * 0.3).astype(jnp.int32), 5)
    k = jnp.minimum(k, mask_len * mask_len)

    for z_ref, m_ref in ((zs_ref, ms_ref), (ze_ref, me_ref)):
        p = jax.nn.sigmoid(z_ref[0])
        keys = jax.lax.bitcast_convert_type(p, jnp.int32)

        def body(_, lohi):
            lo, hi = lohi
            mid = lo + (hi - lo) // 2
            cnt = jnp.sum((keys >= mid).astype(jnp.int32))
            big = cnt >= k
            return jnp.where(big, mid, lo), jnp.where(big, hi, mid)

        lo, _ = jax.lax.fori_loop(
            0, 31, body, (jnp.int32(0), jnp.int32(_ONE_BITS_P1)))
        m_ref[...] = (keys >= lo).astype(jnp.float32)[None]

    @pl.when(b == 0)
    def _():
        ls_ref[0, 0] = jnp.sum(accs_ref[...]) * (1.0 / _N)
        le_ref[0, 0] = jnp.sum(acce_ref[...]) * (1.0 / _N)


def kernel(table, attention_mask, table_labels_S, table_labels_E, W_S, b_S, W_E, b_E):
    x = table.reshape(_N, _D)
    ys3 = table_labels_S.reshape(_NCHUNK, 1, _CHUNK)
    ye3 = table_labels_E.reshape(_NCHUNK, 1, _CHUNK)
    w2 = jnp.zeros((_D, 8), jnp.float32)
    w2 = w2.at[:, 0].set(W_S[:, 0]).at[:, 1].set(W_E[:, 0])
    w8 = w2.astype(jnp.bfloat16)
    bias = jnp.concatenate([b_S, b_E])

    zs, ze, accs, acce = pl.pallas_call(
        _phase1_body,
        grid=(_NCHUNK,),
        in_specs=[
            pl.BlockSpec((_D, 8), lambda i: (0, 0)),
            pl.BlockSpec(memory_space=pltpu.SMEM),
            pl.BlockSpec((_CHUNK, _D), lambda i: (i, 0)),
            pl.BlockSpec((1, 1, _CHUNK), lambda i: (i, 0, 0)),
            pl.BlockSpec((1, 1, _CHUNK), lambda i: (i, 0, 0)),
        ],
        out_specs=[
            pl.BlockSpec((1, 1, _CHUNK), lambda i: (i, 0, 0)),
            pl.BlockSpec((1, 1, _CHUNK), lambda i: (i, 0, 0)),
            pl.BlockSpec((1, 1, _CHUNK), lambda i: (0, 0, 0)),
            pl.BlockSpec((1, 1, _CHUNK), lambda i: (0, 0, 0)),
        ],
        out_shape=[
            jax.ShapeDtypeStruct((_NCHUNK, 1, _CHUNK), jnp.float32),
            jax.ShapeDtypeStruct((_NCHUNK, 1, _CHUNK), jnp.float32),
            jax.ShapeDtypeStruct((1, 1, _CHUNK), jnp.float32),
            jax.ShapeDtypeStruct((1, 1, _CHUNK), jnp.float32),
        ],
    )(w8, bias, x, ys3, ye3)

    am3 = attention_mask.reshape(_B, 1, _L)
    zsb = zs.reshape(_B, _CPB, _CHUNK)
    zeb = ze.reshape(_B, _CPB, _CHUNK)
    ms, me, ls, le = pl.pallas_call(
        _phase2_body,
        grid=(_B,),
        in_specs=[
            pl.BlockSpec((1, 1, _L), lambda b: (b, 0, 0)),
            pl.BlockSpec((1, _CPB, _CHUNK), lambda b: (b, 0, 0)),
            pl.BlockSpec((1, _CPB, _CHUNK), lambda b: (b, 0, 0)),
            pl.BlockSpec((1, 1, _CHUNK), lambda b: (0, 0, 0)),
            pl.BlockSpec((1, 1, _CHUNK), lambda b: (0, 0, 0)),
        ],
        out_specs=[
            pl.BlockSpec((1, _CPB, _CHUNK), lambda b: (b, 0, 0)),
            pl.BlockSpec((1, _CPB, _CHUNK), lambda b: (b, 0, 0)),
            pl.BlockSpec((1, 1), lambda b: (0, 0), memory_space=pltpu.SMEM),
            pl.BlockSpec((1, 1), lambda b: (0, 0), memory_space=pltpu.SMEM),
        ],
        out_shape=[
            jax.ShapeDtypeStruct((_B, _CPB, _CHUNK), jnp.float32),
            jax.ShapeDtypeStruct((_B, _CPB, _CHUNK), jnp.float32),
            jax.ShapeDtypeStruct((1, 1), jnp.float32),
            jax.ShapeDtypeStruct((1, 1), jnp.float32),
        ],
    )(am3, zsb, zeb, accs, acce)

    return (ls[0, 0], le[0, 0],
            ms.reshape(_B, _L, _L).astype(bool),
            me.reshape(_B, _L, _L).astype(bool))


# speedup vs baseline: 1.1383x; 1.1383x over previous
import jax
import jax.numpy as jnp
from jax.experimental import pallas as pl
from jax.experimental.pallas import tpu as pltpu

_B, _L, _D = 4, 128, 768
_N = _B * _L * _L
_CHUNK = 2048
_NCHUNK = _N // _CHUNK
_CPB = _N // _B // _CHUNK
_ONE_BITS_P1 = 0x3F800001


def _phase1_body(w8_ref, bias_ref, x_ref, ys_ref, ye_ref,
                 zs_ref, ze_ref, accs_ref, acce_ref):
    i = pl.program_id(0)
    x = x_ref[...]
    wcat = w8_ref[...]
    x_hi = x.astype(jnp.bfloat16)
    dn = (((1,), (0,)), ((), ()))
    res = jax.lax.dot_general(
        x_hi, wcat, dn,
        preferred_element_type=jnp.float32)
    rt = res.T
    zs = rt[0:1, :] + bias_ref[0]
    ze = rt[1:2, :] + bias_ref[1]
    ys_i = ys_ref[0]
    ye_i = ye_ref[0]
    ys = ys_i.astype(jnp.float32)
    ye = ye_i.astype(jnp.float32)
    w = (ys_i >= 0).astype(jnp.float32)
    bce_s = jnp.maximum(zs, 0.0) - zs * ys + jnp.log1p(jnp.exp(-jnp.abs(zs)))
    bce_e = jnp.maximum(ze, 0.0) - ze * ye + jnp.log1p(jnp.exp(-jnp.abs(ze)))

    @pl.when(i == 0)
    def _():
        accs_ref[...] = jnp.zeros_like(accs_ref)
        acce_ref[...] = jnp.zeros_like(acce_ref)

    accs_ref[...] += (w * bce_s)[None]
    acce_ref[...] += (w * bce_e)[None]
    neg = jnp.float32(-jnp.inf)
    zs_ref[...] = jnp.where(w > 0, zs, neg)[None]
    ze_ref[...] = jnp.where(w > 0, ze, neg)[None]


def _phase2_body(am_ref, zs_ref, ze_ref, accs_ref, acce_ref,
                 ms_ref, me_ref, ls_ref, le_ref):
    b = pl.program_id(0)
    mask_len = jnp.sum(am_ref[...]) - 2
    k = jnp.maximum((mask_len.astype(jnp.float32) * 0.3).astype(jnp.int32), 5)
    k = jnp.minimum(k, mask_len * mask_len)

    for z_ref, m_ref in ((zs_ref, ms_ref), (ze_ref, me_ref)):
        p = jax.nn.sigmoid(z_ref[0])
        keys = jax.lax.bitcast_convert_type(p, jnp.int32)

        def body(_, lohi):
            lo, hi = lohi
            mid = lo + (hi - lo) // 2
            cnt = jnp.sum((keys >= mid).astype(jnp.int32))
            big = cnt >= k
            return jnp.where(big, mid, lo), jnp.where(big, hi, mid)

        lo, _ = jax.lax.fori_loop(
            0, 31, body, (jnp.int32(0), jnp.int32(_ONE_BITS_P1)))
        m_ref[...] = (keys >= lo).astype(jnp.float32)[None]

    @pl.when(b == 0)
    def _():
        ls_ref[0, 0] = jnp.sum(accs_ref[...]) * (1.0 / _N)
        le_ref[0, 0] = jnp.sum(acce_ref[...]) * (1.0 / _N)


def kernel(table, attention_mask, table_labels_S, table_labels_E, W_S, b_S, W_E, b_E):
    x = table.reshape(_N, _D)
    ys3 = table_labels_S.reshape(_NCHUNK, 1, _CHUNK)
    ye3 = table_labels_E.reshape(_NCHUNK, 1, _CHUNK)
    w2 = jnp.zeros((_D, 8), jnp.float32)
    w2 = w2.at[:, 0].set(W_S[:, 0]).at[:, 1].set(W_E[:, 0])
    w8 = w2.astype(jnp.bfloat16)
    bias = jnp.concatenate([b_S, b_E])

    zs, ze, accs, acce = pl.pallas_call(
        _phase1_body,
        grid=(_NCHUNK,),
        in_specs=[
            pl.BlockSpec((_D, 8), lambda i: (0, 0)),
            pl.BlockSpec(memory_space=pltpu.SMEM),
            pl.BlockSpec((_CHUNK, _D), lambda i: (i, 0)),
            pl.BlockSpec((1, 1, _CHUNK), lambda i: (i, 0, 0)),
            pl.BlockSpec((1, 1, _CHUNK), lambda i: (i, 0, 0)),
        ],
        out_specs=[
            pl.BlockSpec((1, 1, _CHUNK), lambda i: (i, 0, 0)),
            pl.BlockSpec((1, 1, _CHUNK), lambda i: (i, 0, 0)),
            pl.BlockSpec((1, 1, _CHUNK), lambda i: (0, 0, 0)),
            pl.BlockSpec((1, 1, _CHUNK), lambda i: (0, 0, 0)),
        ],
        out_shape=[
            jax.ShapeDtypeStruct((_NCHUNK, 1, _CHUNK), jnp.float32),
            jax.ShapeDtypeStruct((_NCHUNK, 1, _CHUNK), jnp.float32),
            jax.ShapeDtypeStruct((1, 1, _CHUNK), jnp.float32),
            jax.ShapeDtypeStruct((1, 1, _CHUNK), jnp.float32),
        ],
    )(w8, bias, x, ys3, ye3)

    am3 = attention_mask.reshape(_B, 1, _L)
    zsb = zs.reshape(_B, _CPB, _CHUNK)
    zeb = ze.reshape(_B, _CPB, _CHUNK)
    ms, me, ls, le = pl.pallas_call(
        _phase2_body,
        grid=(_B,),
        in_specs=[
            pl.BlockSpec((1, 1, _L), lambda b: (b, 0, 0)),
            pl.BlockSpec((1, _CPB, _CHUNK), lambda b: (b, 0, 0)),
            pl.BlockSpec((1, _CPB, _CHUNK), lambda b: (b, 0, 0)),
            pl.BlockSpec((1, 1, _CHUNK), lambda b: (0, 0, 0)),
            pl.BlockSpec((1, 1, _CHUNK), lambda b: (0, 0, 0)),
        ],
        out_specs=[
            pl.BlockSpec((1, _CPB, _CHUNK), lambda b: (b, 0, 0)),
            pl.BlockSpec((1, _CPB, _CHUNK), lambda b: (b, 0, 0)),
            pl.BlockSpec((1, 1), lambda b: (0, 0), memory_space=pltpu.SMEM),
            pl.BlockSpec((1, 1), lambda b: (0, 0), memory_space=pltpu.SMEM),
        ],
        out_shape=[
            jax.ShapeDtypeStruct((_B, _CPB, _CHUNK), jnp.float32),
            jax.ShapeDtypeStruct((_B, _CPB, _CHUNK), jnp.float32),
            jax.ShapeDtypeStruct((1, 1), jnp.float32),
            jax.ShapeDtypeStruct((1, 1), jnp.float32),
        ],
    )(am3, zsb, zeb, accs, acce)

    return (ls[0, 0], le[0, 0],
            ms.reshape(_B, _L, _L).astype(bool),
            me.reshape(_B, _L, _L).astype(bool))
